# Initial kernel scaffold; baseline (speedup 1.0000x reference)
import jax
import jax.numpy as jnp
from jax.experimental import pallas as pl


def kernel(x, encoding):
    raise NotImplementedError("write your pallas kernel here")


# speedup vs baseline: 7.3698x; 7.3698x over previous
import jax
import jax.numpy as jnp
from jax.experimental import pallas as pl


def _body(enc_ref, out_ref):
    out_ref[...] = jnp.broadcast_to(enc_ref[...][None], out_ref.shape)


def kernel(x, encoding):
    B, S, D = x.shape
    bs = 256
    out = pl.pallas_call(
        _body,
        grid=(S // bs,),
        in_specs=[pl.BlockSpec((bs, D), lambda i: (i, 0))],
        out_specs=pl.BlockSpec((B, bs, D), lambda i: (0, i, 0)),
        out_shape=jax.ShapeDtypeStruct((B, S, D), jnp.float32),
    )(encoding[:S])
    return out
